# baseline (speedup 1.0000x reference)
import functools

import jax
import jax.numpy as jnp
from jax import lax
from jax.experimental import pallas as pl
from jax.experimental.pallas import tpu as pltpu
from jax.experimental.pallas import tpu_sc as plsc

VOCAB = 100000
HID = 128
SEQ = 200
BATCH = 1024

NUM_CORES = 2
NUM_SUBCORES = 16
NW = NUM_CORES * NUM_SUBCORES
NROWS = SEQ * BATCH
ROWS_PER_W = NROWS // NW
GROUP = 128
GROUPS_PER_W = ROWS_PER_W // GROUP
SUPER = 2
SROWS = SUPER * GROUP
NSLOT = 3
SUPERS = GROUPS_PER_W // SUPER

_mesh = plsc.VectorSubcoreMesh(
    core_axis_name="c", subcore_axis_name="s",
    num_cores=NUM_CORES, num_subcores=NUM_SUBCORES,
)


@functools.partial(
    pl.kernel,
    out_type=jax.ShapeDtypeStruct((NROWS, HID), jnp.float32),
    mesh=_mesh,
    scratch_types=[
        pltpu.VMEM((GROUPS_PER_W, GROUP), jnp.int32),
        pltpu.VMEM((NSLOT, SROWS, HID), jnp.float32),
        pltpu.SemaphoreType.DMA((NSLOT,)),
        pltpu.SemaphoreType.DMA((NSLOT,)),
    ],
)
def _gather_kernel(table_hbm, idx_hbm, out_hbm, idx_v, rows_v, gsem, ssem):
    wid = lax.axis_index("s") * NUM_CORES + lax.axis_index("c")
    row_base = wid * ROWS_PER_W

    pltpu.sync_copy(idx_hbm.at[wid], idx_v)

    def issue_gather(s, slot):
        for j in range(SUPER):
            pltpu.async_copy(
                table_hbm.at[idx_v.at[s * SUPER + j]],
                rows_v.at[slot].at[pl.ds(j * GROUP, GROUP)],
                gsem.at[slot],
            )

    def wait_gather(slot):
        pltpu.make_async_copy(
            table_hbm.at[idx_v.at[0]], rows_v.at[slot], gsem.at[slot]
        ).wait()

    def issue_store(s, slot):
        pltpu.async_copy(
            rows_v.at[slot],
            out_hbm.at[pl.ds(row_base + s * SROWS, SROWS)],
            ssem.at[slot],
        )

    def wait_store(s, slot):
        pltpu.make_async_copy(
            rows_v.at[slot],
            out_hbm.at[pl.ds(row_base + s * SROWS, SROWS)],
            ssem.at[slot],
        ).wait()

    issue_gather(0, 0)

    def body(s, carry):
        slot = lax.rem(s, NSLOT)
        sg = lax.rem(s + 1, NSLOT)

        @pl.when(jnp.logical_and(s >= NSLOT - 1, s + 1 < SUPERS))
        def _():
            wait_store(s - (NSLOT - 1), sg)

        @pl.when(s + 1 < SUPERS)
        def _():
            issue_gather(s + 1, sg)

        wait_gather(slot)
        issue_store(s, slot)
        return carry

    lax.fori_loop(0, SUPERS, body, 0)

    for ss in range(SUPERS - NSLOT, SUPERS):
        wait_store(ss, ss % NSLOT)


def kernel(node_embeddings, src_rids):
    idx = jnp.transpose(src_rids, (1, 0, 2)).reshape(NW, GROUPS_PER_W, GROUP)
    out = _gather_kernel(node_embeddings, idx)
    return out.reshape(BATCH, SEQ, HID)


# speedup vs baseline: 1.0076x; 1.0076x over previous
import functools

import jax
import jax.numpy as jnp
from jax import lax
from jax.experimental import pallas as pl
from jax.experimental.pallas import tpu as pltpu
from jax.experimental.pallas import tpu_sc as plsc

VOCAB = 100000
HID = 128
SEQ = 200
BATCH = 1024

NUM_CORES = 2
NUM_SUBCORES = 16
NW = NUM_CORES * NUM_SUBCORES
NROWS = SEQ * BATCH
ROWS_PER_W = NROWS // NW
GROUP = 128
GROUPS_PER_W = ROWS_PER_W // GROUP
NBUF = 5
NROUNDS = GROUPS_PER_W // NBUF

_mesh = plsc.VectorSubcoreMesh(
    core_axis_name="c", subcore_axis_name="s",
    num_cores=NUM_CORES, num_subcores=NUM_SUBCORES,
)


@functools.partial(
    pl.kernel,
    out_type=jax.ShapeDtypeStruct((NROWS, HID), jnp.float32),
    mesh=_mesh,
    scratch_types=[
        pltpu.VMEM((GROUPS_PER_W, GROUP), jnp.int32),
        pltpu.VMEM((NBUF, GROUP, HID), jnp.float32),
        pltpu.SemaphoreType.DMA((NBUF,)),
        pltpu.SemaphoreType.DMA((NBUF,)),
    ],
)
def _gather_kernel(table_hbm, idx_hbm, out_hbm, idx_v, rows_v, gsem, ssem):
    wid = lax.axis_index("s") * NUM_CORES + lax.axis_index("c")
    grp_base = wid * GROUPS_PER_W
    pltpu.sync_copy(idx_hbm.at[wid], idx_v)

    LEAD = 2

    for b in range(LEAD):
        pltpu.async_copy(table_hbm.at[idx_v.at[b]], rows_v.at[b], gsem.at[b])

    def body(i, carry):
        for b in range(NBUF):
            g = i * NBUF + b
            sg = (b + LEAD) % NBUF

            @pl.when(jnp.logical_and(g >= NBUF - LEAD,
                                     g + LEAD < GROUPS_PER_W))
            def _():
                prev_off = (grp_base + g - (NBUF - LEAD)) * GROUP
                pltpu.make_async_copy(
                    rows_v.at[sg],
                    out_hbm.at[pl.ds(prev_off, GROUP)],
                    ssem.at[sg],
                ).wait()

            @pl.when(g + LEAD < GROUPS_PER_W)
            def _():
                pltpu.async_copy(
                    table_hbm.at[idx_v.at[g + LEAD]], rows_v.at[sg],
                    gsem.at[sg],
                )

            pltpu.make_async_copy(
                table_hbm.at[idx_v.at[b]], rows_v.at[b], gsem.at[b]
            ).wait()
            row_off = (grp_base + g) * GROUP
            pltpu.async_copy(
                rows_v.at[b], out_hbm.at[pl.ds(row_off, GROUP)], ssem.at[b]
            )

        return carry

    lax.fori_loop(0, NROUNDS, body, 0)

    for gg in range(GROUPS_PER_W - NBUF, GROUPS_PER_W):
        b = gg % NBUF
        row_off = (grp_base + gg) * GROUP
        pltpu.make_async_copy(
            rows_v.at[b], out_hbm.at[pl.ds(row_off, GROUP)], ssem.at[b]
        ).wait()


def kernel(node_embeddings, src_rids):
    idx = jnp.transpose(src_rids, (1, 0, 2)).reshape(NW, GROUPS_PER_W, GROUP)
    out = _gather_kernel(node_embeddings, idx)
    return out.reshape(BATCH, SEQ, HID)
